# baseline (speedup 1.0000x reference)
import functools

import jax
import jax.numpy as jnp
from jax.experimental import pallas as pl
from jax.experimental.pallas import tpu as pltpu


def _dup_matrix(w, dtype):
    c = jnp.arange(4 * w)
    src = (c % (2 * w)) // 2
    return (jnp.arange(w)[:, None] == src[None, :]).astype(dtype)


def _up2x_kernel(x_ref, e_ref, o_ref):
    o_ref[...] = jnp.dot(
        x_ref[...], e_ref[...], preferred_element_type=jnp.float32
    ).astype(o_ref.dtype)


def _up2x_rows(x2, block_rows):
    rows, w = x2.shape
    dt = x2.dtype
    e = _dup_matrix(w, dt)
    br = min(block_rows, rows)
    out = pl.pallas_call(
        _up2x_kernel,
        out_shape=jax.ShapeDtypeStruct((rows, 4 * w), dt),
        grid=(pl.cdiv(rows, br),),
        in_specs=[
            pl.BlockSpec((br, w), lambda i: (i, 0)),
            pl.BlockSpec((w, 4 * w), lambda i: (0, 0)),
        ],
        out_specs=pl.BlockSpec((br, 4 * w), lambda i: (i, 0)),
        compiler_params=pltpu.CompilerParams(
            dimension_semantics=("parallel",),
            vmem_limit_bytes=48 << 20,
        ),
    )(x2, e)
    return out.reshape(2 * rows, 2 * w)


def kernel(x):
    n, c, h, w = x.shape
    x2 = x.reshape(n * c * h, w)
    out2 = _up2x_rows(x2, 2048)
    return out2.reshape(n, c, 2 * h, 2 * w)


# speedup vs baseline: 2.2403x; 2.2403x over previous
import jax
import jax.numpy as jnp
from jax.experimental import pallas as pl
from jax.experimental.pallas import tpu as pltpu


def _half_dup_matrices(w, dtype):
    l = jnp.arange(2)[:, None, None]
    i = jnp.arange(w)[None, :, None]
    j = jnp.arange(w)[None, None, :]
    return (i == l * (w // 2) + j // 2).astype(dtype)


def _up2x_kernel(x_ref, e_ref, o_ref):
    y = jnp.dot(x_ref[...], e_ref[0], preferred_element_type=jnp.float32)
    y = y.astype(o_ref.dtype)
    br = y.shape[0]
    o_ref[pl.ds(0, br, 2), :] = y
    o_ref[pl.ds(1, br, 2), :] = y


def _up2x_rows(x2, block_rows):
    rows, w = x2.shape
    dt = x2.dtype
    e = _half_dup_matrices(w, dt)
    br = min(block_rows, rows)
    return pl.pallas_call(
        _up2x_kernel,
        out_shape=jax.ShapeDtypeStruct((2 * rows, 2 * w), dt),
        grid=(pl.cdiv(rows, br), 2),
        in_specs=[
            pl.BlockSpec((br, w), lambda i, l: (i, 0)),
            pl.BlockSpec((1, w, w), lambda i, l: (l, 0, 0)),
        ],
        out_specs=pl.BlockSpec((2 * br, w), lambda i, l: (i, l)),
        compiler_params=pltpu.CompilerParams(
            dimension_semantics=("parallel", "arbitrary"),
            vmem_limit_bytes=48 << 20,
        ),
    )(x2, e)


def kernel(x):
    n, c, h, w = x.shape
    x2 = x.reshape(n * c * h, w)
    out2 = _up2x_rows(x2, 2048)
    return out2.reshape(n, c, 2 * h, 2 * w)


# speedup vs baseline: 3.0464x; 1.3598x over previous
import jax
import jax.numpy as jnp
from jax.experimental import pallas as pl
from jax.experimental.pallas import tpu as pltpu


def _half_dup_matrices(w, dtype):
    l = jnp.arange(2)[:, None, None]
    i = jnp.arange(w)[None, :, None]
    j = jnp.arange(w)[None, None, :]
    return (i == l * (w // 2) + j // 2).astype(dtype)


def _up2x_kernel(x_ref, e_ref, o_ref):
    y = jnp.dot(x_ref[...], e_ref[0], preferred_element_type=jnp.float32)
    y = y.astype(o_ref.dtype)
    br = y.shape[0]
    o_ref[pl.ds(0, br, 2), :] = y
    o_ref[pl.ds(1, br, 2), :] = y


def _up2x_rows(x2, block_rows):
    rows, w = x2.shape
    dt = x2.dtype
    e = _half_dup_matrices(w, dt)
    br = min(block_rows, rows)
    return pl.pallas_call(
        _up2x_kernel,
        out_shape=jax.ShapeDtypeStruct((2 * rows, 2 * w), dt),
        grid=(pl.cdiv(rows, br), 2),
        in_specs=[
            pl.BlockSpec((br, w), lambda i, l: (i, 0)),
            pl.BlockSpec((1, w, w), lambda i, l: (l, 0, 0)),
        ],
        out_specs=pl.BlockSpec((2 * br, w), lambda i, l: (i, l)),
        compiler_params=pltpu.CompilerParams(
            dimension_semantics=("parallel", "arbitrary"),
            vmem_limit_bytes=48 << 20,
        ),
    )(x2, e)


def kernel(x):
    n, c, h, w = x.shape
    x2 = x.reshape(n * c * h, w)
    out2 = _up2x_rows(x2, 4096)
    return out2.reshape(n, c, 2 * h, 2 * w)


# speedup vs baseline: 3.6449x; 1.1965x over previous
import jax
import jax.numpy as jnp
from jax.experimental import pallas as pl
from jax.experimental.pallas import tpu as pltpu


def _half_dup_matrices(w, dtype):
    l = jnp.arange(2)[:, None, None]
    i = jnp.arange(w)[None, :, None]
    j = jnp.arange(w)[None, None, :]
    return (i == l * (w // 2) + j // 2).astype(dtype)


def _up2x_kernel(x_ref, e_ref, o_ref):
    y = jnp.dot(x_ref[...], e_ref[0], preferred_element_type=jnp.float32)
    y = y.astype(o_ref.dtype)
    br = y.shape[0]
    o_ref[pl.ds(0, br, 2), :] = y
    o_ref[pl.ds(1, br, 2), :] = y


def _up2x_rows(x2, block_rows):
    rows, w = x2.shape
    dt = x2.dtype
    e = _half_dup_matrices(w, dt)
    br = min(block_rows, rows)
    return pl.pallas_call(
        _up2x_kernel,
        out_shape=jax.ShapeDtypeStruct((2 * rows, 2 * w), dt),
        grid=(pl.cdiv(rows, br), 2),
        in_specs=[
            pl.BlockSpec((br, w), lambda i, l: (i, 0)),
            pl.BlockSpec((1, w, w), lambda i, l: (l, 0, 0)),
        ],
        out_specs=pl.BlockSpec((2 * br, w), lambda i, l: (i, l)),
        compiler_params=pltpu.CompilerParams(
            dimension_semantics=("parallel", "arbitrary"),
            vmem_limit_bytes=48 << 20,
        ),
    )(x2, e)


def kernel(x):
    n, c, h, w = x.shape
    x2 = x.reshape(n * c * h, w)
    out2 = _up2x_rows(x2, 8192)
    return out2.reshape(n, c, 2 * h, 2 * w)
